# baseline (speedup 1.0000x reference)
import functools

import numpy as np
import jax
import jax.numpy as jnp
from jax import lax
from jax.experimental import pallas as pl
from jax.experimental.pallas import tpu as pltpu
from jax.experimental.pallas import tpu_sc as plsc

_CHUNK = 128
_MLP_BLK = 8192
_L = 16


def _pack_perm(D):
    P = np.empty((D,), dtype=np.int32)
    for c in range(D // (2 * _L)):
        for i in range(_L):
            P[32 * c + 2 * i] = 32 * c + i
            P[32 * c + 2 * i + 1] = 32 * c + _L + i
    return P


@functools.lru_cache(maxsize=None)
def _make_sc_gather_bf16(V, D, B):
    info = plsc.get_sparse_core_info()
    NC, NS = info.num_cores, info.num_subcores
    NW = NC * NS
    b_per_w = B // NW
    n_streams = max(b_per_w // _CHUNK, 1)
    step = min(_CHUNK, b_per_w)
    n_pairs = D // (2 * _L)
    mesh = plsc.VectorSubcoreMesh(core_axis_name="c", subcore_axis_name="s")

    @functools.partial(
        pl.kernel,
        mesh=mesh,
        out_type=jax.ShapeDtypeStruct((B, D // 2), jnp.int32),
        scratch_types=[
            pltpu.VMEM((b_per_w,), jnp.int32),
            pltpu.VMEM((2 * step, D), jnp.float32),
            pltpu.VMEM((b_per_w, D // 2), jnp.int32),
            pltpu.SemaphoreType.DMA,
            pltpu.SemaphoreType.DMA,
        ],
    )
    def gather_k(idx_hbm, table_hbm, out_hbm, idx_v, rows_v, rows_bf, sem_g, sem_w):
        wid = lax.axis_index("s") * NC + lax.axis_index("c")
        base = wid * b_per_w
        pltpu.sync_copy(idx_hbm.at[pl.ds(base, b_per_w)], idx_v)

        def fire_gather(j):
            return pltpu.async_copy(
                table_hbm.at[idx_v.at[pl.ds(j * step, step)]],
                rows_v.at[pl.ds((j % 2) * step, step)],
                sem_g,
            )

        half = jnp.full((_L,), 0x8000, dtype=jnp.int32)
        shift16 = jnp.full((_L,), 16, dtype=jnp.int32)
        topmask = jnp.full((_L,), -0x10000, dtype=jnp.int32)
        gathers = [fire_gather(0), fire_gather(1)] if n_streams > 1 else [fire_gather(0)]
        writes = []
        for j in range(n_streams):
            gathers[j].wait()
            buf = (j % 2) * step

            def row_body(r, carry):
                for c in range(n_pairs):
                    a = lax.bitcast_convert_type(
                        rows_v[buf + r, pl.ds(32 * c, _L)], jnp.int32)
                    b = lax.bitcast_convert_type(
                        rows_v[buf + r, pl.ds(32 * c + _L, _L)], jnp.int32)
                    lo = lax.shift_right_logical(a + half, shift16)
                    hi = (b + half) & topmask
                    rows_bf[j * step + r, pl.ds(_L * c, _L)] = lo | hi
                return carry

            lax.fori_loop(0, step, row_body, 0, unroll=2)
            if j + 2 < n_streams:
                gathers.append(fire_gather(j + 2))
            writes.append(
                pltpu.async_copy(
                    rows_bf.at[pl.ds(j * step, step)],
                    out_hbm.at[pl.ds(base + j * step, step)],
                    sem_w,
                )
            )
        for w in writes:
            w.wait()

    return gather_k


def _mlp_body(x_ref, w1_ref, b1_ref, w2_ref, b2_ref, o_ref):
    x = x_ref[...].astype(jnp.float32)
    h = jnp.dot(x, w1_ref[...], preferred_element_type=jnp.float32)
    h = h + b1_ref[...]
    h = h * jax.nn.sigmoid(h)
    o = jnp.dot(h, w2_ref[...], preferred_element_type=jnp.float32)
    o_ref[...] = o + b2_ref[...]


@functools.lru_cache(maxsize=None)
def _make_mlp(B, D, blk):
    grid = B // blk
    return pl.pallas_call(
        _mlp_body,
        grid=(grid,),
        in_specs=[
            pl.BlockSpec((blk, D), lambda i: (i, 0)),
            pl.BlockSpec((D, D), lambda i: (0, 0)),
            pl.BlockSpec((1, D), lambda i: (0, 0)),
            pl.BlockSpec((D, D), lambda i: (0, 0)),
            pl.BlockSpec((1, D), lambda i: (0, 0)),
        ],
        out_specs=pl.BlockSpec((blk, D), lambda i: (i, 0)),
        out_shape=jax.ShapeDtypeStruct((B, D), jnp.float32),
    )


def kernel(user_indices, table, W1, b1, W2, b2):
    idx = user_indices.astype(jnp.int32)
    V, D = table.shape
    B = idx.shape[0]
    packed = _make_sc_gather_bf16(V, D, B)(idx, table)
    gathered = lax.bitcast_convert_type(packed, jnp.bfloat16).reshape(B, D)
    W1p = W1[_pack_perm(D), :]
    out = _make_mlp(B, D, min(_MLP_BLK, B))(
        gathered, W1p, b1.reshape(1, D), W2, b2.reshape(1, D)
    )
    return out


# speedup vs baseline: 2.3581x; 2.3581x over previous
import functools

import jax
import jax.numpy as jnp
from jax import lax
from jax.experimental import pallas as pl
from jax.experimental.pallas import tpu as pltpu
from jax.experimental.pallas import tpu_sc as plsc

_CHUNK = 128
_NCH = 1
_MLP_BLK = 8192


@functools.lru_cache(maxsize=None)
def _make_sc_gather(V, D, B):
    info = plsc.get_sparse_core_info()
    NC, NS = info.num_cores, info.num_subcores
    NW = NC * NS
    b_per_w = B // NW
    n_streams = max(b_per_w // _CHUNK, 1)
    step = min(_CHUNK, b_per_w)
    mesh = plsc.VectorSubcoreMesh(core_axis_name="c", subcore_axis_name="s")

    @functools.partial(
        pl.kernel,
        mesh=mesh,
        out_type=jax.ShapeDtypeStruct((B, D), jnp.float32),
        scratch_types=[
            pltpu.VMEM((b_per_w,), jnp.int32),
            pltpu.VMEM((b_per_w, D), jnp.float32),
            pltpu.SemaphoreType.DMA,
            pltpu.SemaphoreType.DMA,
        ],
    )
    def gather_k(idx_hbm, table_hbm, out_hbm, idx_v, rows_v, sem_g, sem_w):
        wid = lax.axis_index("s") * NC + lax.axis_index("c")
        base = wid * b_per_w
        pltpu.sync_copy(idx_hbm.at[pl.ds(base, b_per_w)], idx_v)
        gathers = [
            pltpu.async_copy(
                table_hbm.at[idx_v.at[pl.ds(j * step, step)]],
                rows_v.at[pl.ds(j * step, step)],
                sem_g,
            )
            for j in range(n_streams)
        ]
        writes = []
        for j in range(n_streams):
            gathers[j].wait()
            writes.append(
                pltpu.async_copy(
                    rows_v.at[pl.ds(j * step, step)],
                    out_hbm.at[pl.ds(base + j * step, step)],
                    sem_w,
                )
            )
        for w in writes:
            w.wait()

    return gather_k


def _mlp_body(x_ref, w1_ref, b1_ref, w2_ref, b2_ref, o_ref):
    h = jnp.dot(x_ref[...], w1_ref[...], preferred_element_type=jnp.float32)
    h = h + b1_ref[...]
    h = h * jax.nn.sigmoid(h)
    o = jnp.dot(h, w2_ref[...], preferred_element_type=jnp.float32)
    o_ref[...] = o + b2_ref[...]


@functools.lru_cache(maxsize=None)
def _make_mlp(B, D, blk):
    grid = B // blk
    return pl.pallas_call(
        _mlp_body,
        grid=(grid,),
        in_specs=[
            pl.BlockSpec((blk, D), lambda i: (i, 0)),
            pl.BlockSpec((D, D), lambda i: (0, 0)),
            pl.BlockSpec((1, D), lambda i: (0, 0)),
            pl.BlockSpec((D, D), lambda i: (0, 0)),
            pl.BlockSpec((1, D), lambda i: (0, 0)),
        ],
        out_specs=pl.BlockSpec((blk, D), lambda i: (i, 0)),
        out_shape=jax.ShapeDtypeStruct((B, D), jnp.float32),
    )


def kernel(user_indices, table, W1, b1, W2, b2):
    idx = user_indices.astype(jnp.int32)
    V, D = table.shape
    B = idx.shape[0]
    bc = B // _NCH
    b1r = b1.reshape(1, D)
    b2r = b2.reshape(1, D)
    gathered = [
        _make_sc_gather(V, D, bc)(lax.slice(idx, (c * bc,), ((c + 1) * bc,)), table)
        for c in range(_NCH)
    ]
    outs = [
        _make_mlp(bc, D, min(_MLP_BLK, bc))(g, W1, b1r, W2, b2r) for g in gathered
    ]
    return jnp.concatenate(outs, axis=0) if _NCH > 1 else outs[0]


# speedup vs baseline: 2.3851x; 1.0115x over previous
import functools

import jax
import jax.numpy as jnp
from jax import lax
from jax.experimental import pallas as pl
from jax.experimental.pallas import tpu as pltpu
from jax.experimental.pallas import tpu_sc as plsc

_CHUNK = 128
_NCH = 1
_MLP_BLK = 8192


@functools.lru_cache(maxsize=None)
def _make_sc_gather(V, D, B):
    info = plsc.get_sparse_core_info()
    NC, NS = info.num_cores, info.num_subcores
    NW = NC * NS
    b_per_w = B // NW
    n_streams = max(b_per_w // _CHUNK, 1)
    step = min(_CHUNK, b_per_w)
    mesh = plsc.VectorSubcoreMesh(core_axis_name="c", subcore_axis_name="s")

    @functools.partial(
        pl.kernel,
        mesh=mesh,
        out_type=jax.ShapeDtypeStruct((B, D), jnp.float32),
        scratch_types=[
            pltpu.VMEM((b_per_w,), jnp.int32),
            pltpu.VMEM((b_per_w, D), jnp.float32),
            pltpu.SemaphoreType.DMA,
            pltpu.SemaphoreType.DMA,
        ],
    )
    def gather_k(idx_hbm, table_hbm, out_hbm, idx_v, rows_v, sem_g, sem_w):
        wid = lax.axis_index("s") * NC + lax.axis_index("c")
        base = wid * b_per_w
        pltpu.sync_copy(idx_hbm.at[pl.ds(base, b_per_w)], idx_v)
        gathers = [
            pltpu.async_copy(
                table_hbm.at[idx_v.at[pl.ds(j * step, step)]],
                rows_v.at[pl.ds(j * step, step)],
                sem_g,
            )
            for j in range(n_streams)
        ]
        for g in gathers:
            g.wait()
        pltpu.async_copy(rows_v, out_hbm.at[pl.ds(base, b_per_w)], sem_w).wait()

    return gather_k


def _mlp_body(x_ref, w1_ref, b1_ref, w2_ref, b2_ref, o_ref):
    h = jnp.dot(x_ref[...], w1_ref[...], preferred_element_type=jnp.float32)
    h = h + b1_ref[...]
    h = h * jax.nn.sigmoid(h)
    o = jnp.dot(h, w2_ref[...], preferred_element_type=jnp.float32)
    o_ref[...] = o + b2_ref[...]


@functools.lru_cache(maxsize=None)
def _make_mlp(B, D, blk):
    grid = B // blk
    return pl.pallas_call(
        _mlp_body,
        grid=(grid,),
        in_specs=[
            pl.BlockSpec((blk, D), lambda i: (i, 0)),
            pl.BlockSpec((D, D), lambda i: (0, 0)),
            pl.BlockSpec((1, D), lambda i: (0, 0)),
            pl.BlockSpec((D, D), lambda i: (0, 0)),
            pl.BlockSpec((1, D), lambda i: (0, 0)),
        ],
        out_specs=pl.BlockSpec((blk, D), lambda i: (i, 0)),
        out_shape=jax.ShapeDtypeStruct((B, D), jnp.float32),
    )


def kernel(user_indices, table, W1, b1, W2, b2):
    idx = user_indices.astype(jnp.int32)
    V, D = table.shape
    B = idx.shape[0]
    bc = B // _NCH
    b1r = b1.reshape(1, D)
    b2r = b2.reshape(1, D)
    gathered = [
        _make_sc_gather(V, D, bc)(lax.slice(idx, (c * bc,), ((c + 1) * bc,)), table)
        for c in range(_NCH)
    ]
    outs = [
        _make_mlp(bc, D, min(_MLP_BLK, bc))(g, W1, b1r, W2, b2r) for g in gathered
    ]
    return jnp.concatenate(outs, axis=0) if _NCH > 1 else outs[0]


# speedup vs baseline: 2.3927x; 1.0032x over previous
import functools

import jax
import jax.numpy as jnp
from jax import lax
from jax.experimental import pallas as pl
from jax.experimental.pallas import tpu as pltpu
from jax.experimental.pallas import tpu_sc as plsc

_CHUNK = 256
_NCH = 1
_MLP_BLK = 8192


@functools.lru_cache(maxsize=None)
def _make_sc_gather(V, D, B):
    info = plsc.get_sparse_core_info()
    NC, NS = info.num_cores, info.num_subcores
    NW = NC * NS
    b_per_w = B // NW
    n_streams = max(b_per_w // _CHUNK, 1)
    step = min(_CHUNK, b_per_w)
    mesh = plsc.VectorSubcoreMesh(core_axis_name="c", subcore_axis_name="s")

    @functools.partial(
        pl.kernel,
        mesh=mesh,
        out_type=jax.ShapeDtypeStruct((B, D), jnp.float32),
        scratch_types=[
            pltpu.VMEM((b_per_w,), jnp.int32),
            pltpu.VMEM((b_per_w, D), jnp.float32),
            pltpu.SemaphoreType.DMA,
            pltpu.SemaphoreType.DMA,
        ],
    )
    def gather_k(idx_hbm, table_hbm, out_hbm, idx_v, rows_v, sem_g, sem_w):
        wid = lax.axis_index("s") * NC + lax.axis_index("c")
        base = wid * b_per_w
        pltpu.sync_copy(idx_hbm.at[pl.ds(base, b_per_w)], idx_v)
        gathers = [
            pltpu.async_copy(
                table_hbm.at[idx_v.at[pl.ds(j * step, step)]],
                rows_v.at[pl.ds(j * step, step)],
                sem_g,
            )
            for j in range(n_streams)
        ]
        for g in gathers:
            g.wait()
        pltpu.async_copy(rows_v, out_hbm.at[pl.ds(base, b_per_w)], sem_w).wait()

    return gather_k


def _mlp_body(x_ref, w1_ref, b1_ref, w2_ref, b2_ref, o_ref):
    h = jnp.dot(x_ref[...], w1_ref[...], preferred_element_type=jnp.float32)
    h = h + b1_ref[...]
    h = h * jax.nn.sigmoid(h)
    o = jnp.dot(h, w2_ref[...], preferred_element_type=jnp.float32)
    o_ref[...] = o + b2_ref[...]


@functools.lru_cache(maxsize=None)
def _make_mlp(B, D, blk):
    grid = B // blk
    return pl.pallas_call(
        _mlp_body,
        grid=(grid,),
        in_specs=[
            pl.BlockSpec((blk, D), lambda i: (i, 0)),
            pl.BlockSpec((D, D), lambda i: (0, 0)),
            pl.BlockSpec((1, D), lambda i: (0, 0)),
            pl.BlockSpec((D, D), lambda i: (0, 0)),
            pl.BlockSpec((1, D), lambda i: (0, 0)),
        ],
        out_specs=pl.BlockSpec((blk, D), lambda i: (i, 0)),
        out_shape=jax.ShapeDtypeStruct((B, D), jnp.float32),
    )


def kernel(user_indices, table, W1, b1, W2, b2):
    idx = user_indices.astype(jnp.int32)
    V, D = table.shape
    B = idx.shape[0]
    bc = B // _NCH
    b1r = b1.reshape(1, D)
    b2r = b2.reshape(1, D)
    gathered = [
        _make_sc_gather(V, D, bc)(lax.slice(idx, (c * bc,), ((c + 1) * bc,)), table)
        for c in range(_NCH)
    ]
    outs = [
        _make_mlp(bc, D, min(_MLP_BLK, bc))(g, W1, b1r, W2, b2r) for g in gathered
    ]
    return jnp.concatenate(outs, axis=0) if _NCH > 1 else outs[0]


# speedup vs baseline: 2.3937x; 1.0004x over previous
import functools

import jax
import jax.numpy as jnp
from jax import lax
from jax.experimental import pallas as pl
from jax.experimental.pallas import tpu as pltpu
from jax.experimental.pallas import tpu_sc as plsc

_CHUNK = 512
_NCH = 1
_MLP_BLK = 8192


@functools.lru_cache(maxsize=None)
def _make_sc_gather(V, D, B):
    info = plsc.get_sparse_core_info()
    NC, NS = info.num_cores, info.num_subcores
    NW = NC * NS
    b_per_w = B // NW
    n_streams = max(b_per_w // _CHUNK, 1)
    step = min(_CHUNK, b_per_w)
    mesh = plsc.VectorSubcoreMesh(core_axis_name="c", subcore_axis_name="s")

    @functools.partial(
        pl.kernel,
        mesh=mesh,
        out_type=jax.ShapeDtypeStruct((B, D), jnp.float32),
        scratch_types=[
            pltpu.VMEM((b_per_w,), jnp.int32),
            pltpu.VMEM((b_per_w, D), jnp.float32),
            pltpu.SemaphoreType.DMA,
            pltpu.SemaphoreType.DMA,
        ],
    )
    def gather_k(idx_hbm, table_hbm, out_hbm, idx_v, rows_v, sem_g, sem_w):
        wid = lax.axis_index("s") * NC + lax.axis_index("c")
        base = wid * b_per_w
        pltpu.sync_copy(idx_hbm.at[pl.ds(base, b_per_w)], idx_v)
        gathers = [
            pltpu.async_copy(
                table_hbm.at[idx_v.at[pl.ds(j * step, step)]],
                rows_v.at[pl.ds(j * step, step)],
                sem_g,
            )
            for j in range(n_streams)
        ]
        for g in gathers:
            g.wait()
        pltpu.async_copy(rows_v, out_hbm.at[pl.ds(base, b_per_w)], sem_w).wait()

    return gather_k


def _mlp_body(x_ref, w1_ref, b1_ref, w2_ref, b2_ref, o_ref):
    h = jnp.dot(x_ref[...], w1_ref[...], preferred_element_type=jnp.float32)
    h = h + b1_ref[...]
    h = h * jax.nn.sigmoid(h)
    o = jnp.dot(h, w2_ref[...], preferred_element_type=jnp.float32)
    o_ref[...] = o + b2_ref[...]


@functools.lru_cache(maxsize=None)
def _make_mlp(B, D, blk):
    grid = B // blk
    return pl.pallas_call(
        _mlp_body,
        grid=(grid,),
        in_specs=[
            pl.BlockSpec((blk, D), lambda i: (i, 0)),
            pl.BlockSpec((D, D), lambda i: (0, 0)),
            pl.BlockSpec((1, D), lambda i: (0, 0)),
            pl.BlockSpec((D, D), lambda i: (0, 0)),
            pl.BlockSpec((1, D), lambda i: (0, 0)),
        ],
        out_specs=pl.BlockSpec((blk, D), lambda i: (i, 0)),
        out_shape=jax.ShapeDtypeStruct((B, D), jnp.float32),
    )


def kernel(user_indices, table, W1, b1, W2, b2):
    idx = user_indices.astype(jnp.int32)
    V, D = table.shape
    B = idx.shape[0]
    bc = B // _NCH
    b1r = b1.reshape(1, D)
    b2r = b2.reshape(1, D)
    gathered = [
        _make_sc_gather(V, D, bc)(lax.slice(idx, (c * bc,), ((c + 1) * bc,)), table)
        for c in range(_NCH)
    ]
    outs = [
        _make_mlp(bc, D, min(_MLP_BLK, bc))(g, W1, b1r, W2, b2r) for g in gathered
    ]
    return jnp.concatenate(outs, axis=0) if _NCH > 1 else outs[0]
